# baseline (speedup 1.0000x reference)
import functools

import jax
import jax.numpy as jnp
from jax import lax
from jax.experimental import pallas as pl
from jax.experimental.pallas import tpu as pltpu
from jax.experimental.pallas import tpu_sc as plsc

_ROWS = 8192
_D = 768
_NC = 2
_NS = 16
_NW = _NC * _NS
_RPW = _ROWS // _NW
_CH = 64
_NCH = _RPW // _CH
_NBUF = 2
_SC_B = 2
_TC_B = 2
_RB = 512

_mesh = plsc.VectorSubcoreMesh(core_axis_name="c", subcore_axis_name="s")


@functools.partial(
    pl.kernel,
    out_type=jax.ShapeDtypeStruct((_SC_B, _ROWS, _D), jnp.float32),
    mesh=_mesh,
    scratch_types=[
        pltpu.VMEM((_NBUF, _CH, _D), jnp.float32),
    ] + [pltpu.SemaphoreType.DMA] * (2 * _NBUF),
)
def _sc_tile_copy(emb_hbm, out_hbm, bufs, *sems):
    rsems = sems[:_NBUF]
    wsems = sems[_NBUF:]
    wid = lax.axis_index("s") * _NC + lax.axis_index("c")
    base = wid * _RPW

    def rd(i):
        return pltpu.make_async_copy(
            emb_hbm.at[pl.ds(base + i * _CH, _CH)], bufs.at[i % _NBUF],
            rsems[i % _NBUF])

    def wr(i, b):
        return pltpu.make_async_copy(
            bufs.at[i % _NBUF], out_hbm.at[b, pl.ds(base + i * _CH, _CH)],
            wsems[i % _NBUF])

    rd(0).start()
    rd(1).start()
    for i in range(_NCH):
        rd(i).wait()
        for b in range(_SC_B):
            wr(i, b).start()
        nxt = i + 2
        if nxt < _NCH:
            prev = nxt - _NBUF
            if prev >= 0:
                for b in range(_SC_B):
                    wr(prev, b).wait()
            rd(nxt).start()
    for i in range(max(0, _NCH - _NBUF), _NCH):
        for b in range(_SC_B):
            wr(i, b).wait()


def _tc_copy_body(emb_ref, out_ref):
    out_ref[0] = emb_ref[...]


def _tc_copy(emb):
    return pl.pallas_call(
        _tc_copy_body,
        grid=(_ROWS // _RB, _TC_B),
        in_specs=[pl.BlockSpec((_RB, _D), lambda i, b: (i, 0))],
        out_specs=pl.BlockSpec((1, _RB, _D), lambda i, b: (b, i, 0)),
        out_shape=jax.ShapeDtypeStruct((_TC_B, _ROWS, _D), emb.dtype),
    )(emb)


def kernel(B, T, emb):
    del B, T
    sc_half = _sc_tile_copy(emb)
    tc_half = _tc_copy(emb)
    return jnp.concatenate([sc_half, tc_half], axis=0)


# speedup vs baseline: 2.2201x; 2.2201x over previous
import functools

import jax
import jax.numpy as jnp
from jax import lax
from jax.experimental import pallas as pl
from jax.experimental.pallas import tpu as pltpu
from jax.experimental.pallas import tpu_sc as plsc

_ROWS = 8192
_D = 768
_BATCH = 4
_NC = 2
_NS = 16
_NW = _NC * _NS
_RPW = _ROWS // _NW
_CH = 64
_NCH = _RPW // _CH
_NBUF = 2

_mesh = plsc.VectorSubcoreMesh(core_axis_name="c", subcore_axis_name="s")


@functools.partial(
    pl.kernel,
    out_type=jax.ShapeDtypeStruct((_BATCH, _ROWS, _D), jnp.float32),
    mesh=_mesh,
    scratch_types=[
        pltpu.VMEM((_NBUF, _CH, _D), jnp.float32),
    ] + [pltpu.SemaphoreType.DMA] * (2 * _NBUF),
)
def _sc_tile_copy(emb_hbm, out_hbm, bufs, *sems):
    rsems = sems[:_NBUF]
    wsems = sems[_NBUF:]
    wid = lax.axis_index("c") * _NS + lax.axis_index("s")
    base = wid * _RPW

    def rd(i):
        return pltpu.make_async_copy(
            emb_hbm.at[pl.ds(base + i * _CH, _CH)], bufs.at[i % _NBUF],
            rsems[i % _NBUF])

    def wr(i, b):
        return pltpu.make_async_copy(
            bufs.at[i % _NBUF], out_hbm.at[b, pl.ds(base + i * _CH, _CH)],
            wsems[i % _NBUF])

    rd(0).start()
    rd(1).start()
    for i in range(_NCH):
        rd(i).wait()
        for b in range(_BATCH):
            wr(i, b).start()
        nxt = i + 2
        if nxt < _NCH:
            prev = nxt - _NBUF
            if prev >= 0:
                for b in range(_BATCH):
                    wr(prev, b).wait()
            rd(nxt).start()
    for i in range(max(0, _NCH - _NBUF), _NCH):
        for b in range(_BATCH):
            wr(i, b).wait()


def kernel(B, T, emb):
    del B, T
    return _sc_tile_copy(emb)


# speedup vs baseline: 2.2516x; 1.0142x over previous
import functools

import jax
import jax.numpy as jnp
from jax import lax
from jax.experimental import pallas as pl
from jax.experimental.pallas import tpu as pltpu
from jax.experimental.pallas import tpu_sc as plsc

_ROWS = 8192
_D = 768
_BATCH = 4
_NC = 2
_NS = 16
_NW = _NC * _NS
_RPW = _ROWS // _NW
_CH = 128
_NCH = _RPW // _CH
_NBUF = 1

_mesh = plsc.VectorSubcoreMesh(core_axis_name="c", subcore_axis_name="s")


@functools.partial(
    pl.kernel,
    out_type=jax.ShapeDtypeStruct((_BATCH, _ROWS, _D), jnp.float32),
    mesh=_mesh,
    scratch_types=[
        pltpu.VMEM((_NBUF, _CH, _D), jnp.float32),
    ] + [pltpu.SemaphoreType.DMA] * (2 * _NBUF),
)
def _sc_tile_copy(emb_hbm, out_hbm, bufs, *sems):
    rsems = sems[:_NBUF]
    wsems = sems[_NBUF:]
    wid = lax.axis_index("c") * _NS + lax.axis_index("s")
    base = wid * _RPW

    def rd(i):
        return pltpu.make_async_copy(
            emb_hbm.at[pl.ds(base + i * _CH, _CH)], bufs.at[i % _NBUF],
            rsems[i % _NBUF])

    def wr(i, b):
        return pltpu.make_async_copy(
            bufs.at[i % _NBUF], out_hbm.at[b, pl.ds(base + i * _CH, _CH)],
            wsems[i % _NBUF])

    rd(0).start()
    for i in range(_NCH):
        rd(i).wait()
        for b in range(_BATCH):
            wr(i, b).start()
        for b in range(_BATCH):
            wr(i, b).wait()
        if i + 1 < _NCH:
            rd(i + 1).start()


def kernel(B, T, emb):
    del B, T
    return _sc_tile_copy(emb)
